# baseline (speedup 1.0000x reference)
import functools

import jax
import jax.numpy as jnp
from jax import lax
from jax.experimental import pallas as pl
from jax.experimental.pallas import tpu as pltpu
from jax.experimental.pallas import tpu_sc as plsc

B = 16384
F = 26
NV = 200
K = F * NV
NC = 2
NS = 16
NW = NC * NS
CB = B // NW
VPB = CB // 16
RA = 48
RB = 56
_REGIONS = ((0, "a"), (RA, "b"), (RA + RB, "a"), (RA + RB + RA, "b"))


def _make_sc_onehot():
    mesh = plsc.VectorSubcoreMesh(core_axis_name="c", subcore_axis_name="s")

    @functools.partial(
        pl.kernel,
        mesh=mesh,
        out_type=jax.ShapeDtypeStruct((K, B), jnp.float32),
        scratch_types=[
            pltpu.VMEM((F, CB), jnp.int32),
            pltpu.VMEM((RA, CB), jnp.float32),
            pltpu.VMEM((RB, CB), jnp.float32),
            pltpu.VMEM((CB,), jnp.int32),
            pltpu.VMEM((CB,), jnp.int32),
            pltpu.SemaphoreType.DMA,
            pltpu.SemaphoreType.DMA,
            pltpu.SemaphoreType.DMA,
        ],
        compiler_params=pltpu.CompilerParams(needs_layout_passes=False),
    )
    def onehot(xe_hbm, out_hbm, xe_v, buf_a, buf_b, idx_a, idx_b,
               sem_a, sem_b, sem_x):
        wid = lax.axis_index("s") * NC + lax.axis_index("c")
        col0 = wid * CB
        xe_copy = pltpu.async_copy(
            xe_hbm.at[:, pl.ds(col0, CB)], xe_v, sem_x)

        zeros16 = jnp.zeros((16,), jnp.float32)
        ones16 = jnp.ones((16,), jnp.float32)
        iota16 = lax.iota(jnp.int32, 16)

        bufs = {"a": buf_a, "b": buf_b}
        idxs = {"a": idx_a, "b": idx_b}
        sems = {"a": sem_a, "b": sem_b}
        rows = {"a": RA, "b": RB}

        def zero_fill(p):
            def body(i, carry):
                for j in range(CB // 16):
                    bufs[p][i, pl.ds(j * 16, 16)] = zeros16
                return carry
            lax.fori_loop(0, rows[p], body, 0)

        def ones_r(f, v0, p):
            nr = rows[p]
            for k in range(VPB):
                bcol = iota16 + k * 16
                val = plsc.load_gather(xe_v, [jnp.broadcast_to(f, (16,)), bcol])
                row = jnp.minimum(jnp.maximum(val, v0), v0 + nr - 1) - v0
                idxs[p][pl.ds(k * 16, 16)] = row
                plsc.store_scatter(bufs[p], [row, bcol], ones16,
                                   mask=(val >= v0) & (val < v0 + nr))

        def clear_r(p):
            for k in range(VPB):
                bcol = iota16 + k * 16
                row = idxs[p][pl.ds(k * 16, 16)]
                plsc.store_scatter(bufs[p], [row, bcol], zeros16)

        def fire_r(f, v0, p):
            pltpu.async_copy(
                bufs[p],
                out_hbm.at[pl.ds(f * NV + v0, rows[p]), pl.ds(col0, CB)],
                sems[p])

        def drain_r(f, v0, p):
            pltpu.make_async_copy(
                bufs[p],
                out_hbm.at[pl.ds(f * NV + v0, rows[p]), pl.ds(col0, CB)],
                sems[p]).wait()

        xe_copy.wait()
        zero_fill("a")
        ones_r(jnp.int32(0), _REGIONS[0][0], "a")
        fire_r(jnp.int32(0), _REGIONS[0][0], "a")
        zero_fill("b")
        ones_r(jnp.int32(0), _REGIONS[1][0], "b")
        fire_r(jnp.int32(0), _REGIONS[1][0], "b")
        for v0, p in _REGIONS[2:]:
            drain_r(jnp.int32(0), v0 - RA - RB, p)
            clear_r(p)
            ones_r(jnp.int32(0), v0, p)
            fire_r(jnp.int32(0), v0, p)

        def field_body(f, carry):
            for i, (v0, p) in enumerate(_REGIONS):
                pv0, _ = _REGIONS[i - 2]
                pf = f - 1 if i < 2 else f
                drain_r(pf, pv0, p)
                clear_r(p)
                ones_r(f, v0, p)
                fire_r(f, v0, p)
            return carry

        lax.fori_loop(1, F, field_body, 0)
        drain_r(jnp.int32(F - 1), _REGIONS[2][0], "a")
        drain_r(jnp.int32(F - 1), _REGIONS[3][0], "b")

    return onehot


_sc_onehot = _make_sc_onehot()


@jax.jit
def kernel(xe):
    return _sc_onehot(xe.T).T


# speedup vs baseline: 1.1385x; 1.1385x over previous
import functools

import jax
import jax.numpy as jnp
from jax import lax
from jax.experimental import pallas as pl
from jax.experimental.pallas import tpu as pltpu
from jax.experimental.pallas import tpu_sc as plsc

B = 16384
F = 26
NV = 200
K = F * NV
NC = 2
NS = 16
NW = NC * NS
CB = B // NW
VPB = CB // 16
RA = 96
RB = NV - RA


def _make_sc_onehot():
    mesh = plsc.VectorSubcoreMesh(core_axis_name="c", subcore_axis_name="s")

    @functools.partial(
        pl.kernel,
        mesh=mesh,
        out_type=jax.ShapeDtypeStruct((K, B), jnp.float32),
        scratch_types=[
            pltpu.VMEM((F, CB), jnp.int32),
            pltpu.VMEM((RA, CB), jnp.float32),
            pltpu.VMEM((RB, CB), jnp.float32),
            pltpu.VMEM((CB,), jnp.int32),
            pltpu.VMEM((CB,), jnp.int32),
            pltpu.SemaphoreType.DMA,
            pltpu.SemaphoreType.DMA,
            pltpu.SemaphoreType.DMA,
        ],
        compiler_params=pltpu.CompilerParams(needs_layout_passes=False),
    )
    def onehot(xe_hbm, out_hbm, xe_v, buf_a, buf_b, idx_a, idx_b,
               sem_a, sem_b, sem_x):
        wid = lax.axis_index("s") * NC + lax.axis_index("c")
        col0 = wid * CB
        for f in range(F):
            pltpu.async_copy(
                xe_hbm.at[pl.ds(f * B + col0, CB)], xe_v.at[f], sem_x)

        zeros16 = jnp.zeros((16,), jnp.float32)
        ones16 = jnp.ones((16,), jnp.float32)
        iota16 = lax.iota(jnp.int32, 16)

        def zero_a(i, carry):
            for j in range(CB // 16):
                buf_a[i, pl.ds(j * 16, 16)] = zeros16
            return carry

        def zero_b(i, carry):
            for j in range(CB // 16):
                buf_b[i, pl.ds(j * 16, 16)] = zeros16
            return carry

        def ones_a(f):
            for k in range(VPB):
                bcol = iota16 + k * 16
                val = xe_v[f, pl.ds(k * 16, 16)]
                row = jnp.minimum(val, RA - 1)
                idx_a[pl.ds(k * 16, 16)] = row
                plsc.store_scatter(buf_a, [row, bcol], ones16, mask=val < RA)

        def ones_b(f):
            for k in range(VPB):
                bcol = iota16 + k * 16
                val = xe_v[f, pl.ds(k * 16, 16)]
                row = jnp.maximum(val, RA) - RA
                idx_b[pl.ds(k * 16, 16)] = row
                plsc.store_scatter(buf_b, [row, bcol], ones16,
                                   mask=val >= RA)

        def clear_a():
            for k in range(VPB):
                bcol = iota16 + k * 16
                row = idx_a[pl.ds(k * 16, 16)]
                plsc.store_scatter(buf_a, [row, bcol], zeros16)

        def clear_b():
            for k in range(VPB):
                bcol = iota16 + k * 16
                row = idx_b[pl.ds(k * 16, 16)]
                plsc.store_scatter(buf_b, [row, bcol], zeros16)

        def fire_a(f):
            pltpu.async_copy(
                buf_a, out_hbm.at[pl.ds(f * NV, RA), pl.ds(col0, CB)], sem_a)

        def fire_b(f):
            pltpu.async_copy(
                buf_b, out_hbm.at[pl.ds(f * NV + RA, RB), pl.ds(col0, CB)],
                sem_b)

        def drain_a(f):
            pltpu.make_async_copy(
                buf_a, out_hbm.at[pl.ds(f * NV, RA), pl.ds(col0, CB)],
                sem_a).wait()

        def drain_b(f):
            pltpu.make_async_copy(
                buf_b, out_hbm.at[pl.ds(f * NV + RA, RB), pl.ds(col0, CB)],
                sem_b).wait()

        lax.fori_loop(0, RA, zero_a, 0)
        for f in range(F):
            pltpu.make_async_copy(
                xe_hbm.at[pl.ds(f * B + col0, CB)], xe_v.at[f], sem_x).wait()
        ones_a(jnp.int32(0))
        fire_a(jnp.int32(0))
        lax.fori_loop(0, RB, zero_b, 0)
        ones_b(jnp.int32(0))
        fire_b(jnp.int32(0))

        def field_body(f, carry):
            drain_a(f - 1)
            clear_a()
            ones_a(f)
            fire_a(f)
            drain_b(f - 1)
            clear_b()
            ones_b(f)
            fire_b(f)
            return carry

        lax.fori_loop(1, F, field_body, 0)
        drain_a(jnp.int32(F - 1))
        drain_b(jnp.int32(F - 1))

    return onehot


_sc_onehot = _make_sc_onehot()


@jax.jit
def kernel(xe):
    return _sc_onehot(xe.T.reshape(F * B)).T
